# baseline (speedup 1.0000x reference)
import functools

import jax
import jax.numpy as jnp
from jax.experimental import pallas as pl
from jax.experimental.pallas import tpu as pltpu

_CH = 32


def _fold(op, r):
    return op(r, jnp.roll(r, r.shape[-1] // 2, axis=-1))


def _adj_body(K, x_ref, prob_ref, cond_ref, ent_ref, sm_ref):
    V, F = x_ref.shape[1], x_ref.shape[2]
    L = 2 * F
    NCH = V // (2 * _CH)

    def load2(ref, i):
        a = ref[0, pl.ds(i * 2 * _CH, _CH), :]
        b = ref[0, pl.ds(i * 2 * _CH + _CH, _CH), :]
        return jnp.concatenate([a, b], axis=1)

    def store2(ref, i, val):
        ref[0, pl.ds(i * 2 * _CH, _CH), :] = val[:, :F]
        ref[0, pl.ds(i * 2 * _CH + _CH, _CH), :] = val[:, F:]

    lane = jax.lax.broadcasted_iota(jnp.int32, (_CH, L), 1)
    row_iota = (jax.lax.broadcasted_iota(jnp.int32, (_CH, L), 0)
                + _CH * (lane >= F).astype(jnp.int32))

    def pass_a(i, carry):
        acc = carry[0]
        ms = list(carry[1:])
        v = load2(x_ref, i)
        acc = acc + v
        cur = v
        for j in range(K):
            hi = jnp.maximum(ms[j], cur)
            cur = jnp.minimum(ms[j], cur)
            ms[j] = hi
        return (acc, *ms)

    init_a = (jnp.zeros((_CH, L), jnp.float32),) + tuple(
        jnp.full((_CH, L), -jnp.inf, jnp.float32) for _ in range(K))
    res_a = jax.lax.fori_loop(0, NCH, pass_a, init_a)
    s = _fold(jnp.add, jnp.sum(res_a[0], axis=0, keepdims=True))
    cand_x = jnp.concatenate(res_a[1:], axis=0)
    sm_cand = jnp.clip(cand_x / (s + 1e-20), 0.001, 1.0 - 0.001)

    t = jnp.full((1, L), 2.0, jnp.float32)
    n = jnp.zeros((1, L), jnp.int32)
    c = jnp.zeros((1, L), jnp.int32)
    for _ in range(K):
        m = _fold(jnp.maximum,
                  jnp.max(jnp.where(sm_cand < t, sm_cand, -1.0), axis=0,
                          keepdims=True))
        n_new = _fold(jnp.add,
                      jnp.sum((sm_cand >= m).astype(jnp.int32), axis=0,
                              keepdims=True))
        upd = n < K
        c = jnp.where(upd, n, c)
        t = jnp.where(upd, m, t)
        n = jnp.where(upd, n_new, n)
    e = K - c

    def pass_b(i, carry):
        ent_acc = carry[0]
        js = list(carry[1:])
        xv = load2(x_ref, i)
        smv = jnp.clip(xv / (s + 1e-20), 0.001, 1.0 - 0.001)
        store2(sm_ref, i, smv)
        ent_acc = ent_acc - smv * jnp.log(smv)
        cur = jnp.where(smv == t, row_iota + i * 2 * _CH, V)
        for j in range(K):
            lo = jnp.minimum(js[j], cur)
            cur = jnp.maximum(js[j], cur)
            js[j] = lo
        return (ent_acc, *js)

    init_b = (jnp.zeros((_CH, L), jnp.float32),) + tuple(
        jnp.full((_CH, L), V, jnp.int32) for _ in range(K))
    res_b = jax.lax.fori_loop(0, NCH, pass_b, init_b)
    ent_ref[...] = (jnp.sum(res_b[0]) / F).reshape(1, 1, 1)
    cand_i = jnp.concatenate(res_b[1:], axis=0)

    s_last = jnp.full((1, L), -1, jnp.int32)
    last = jnp.full((1, L), -1, jnp.int32)
    for i in range(K):
        cnd = _fold(jnp.minimum,
                    jnp.min(jnp.where(cand_i > last, cand_i, V), axis=0,
                            keepdims=True))
        s_last = jnp.where(i < e, cnd, s_last)
        last = cnd

    thr = 1.0 / (V * K)

    def pass_f(i, carry):
        smv = load2(sm_ref, i)
        rmax = s_last - i * 2 * _CH
        mask = (smv > t) | ((smv == t) & (row_iota <= rmax))
        cond = (mask & (smv > thr)).astype(jnp.int32)
        store2(cond_ref, i, cond)
        store2(prob_ref, i, jnp.where(cond == 1, jnp.log(smv), 0.0))
        return carry

    jax.lax.fori_loop(0, NCH, pass_f, 0)


def kernel(stack_exp):
    B, V, F = stack_exp.shape
    K = 8
    slab = pl.BlockSpec((1, V, F), lambda b: (b, 0, 0))
    prob, cond, ent, sm = pl.pallas_call(
        functools.partial(_adj_body, K),
        grid=(B,),
        in_specs=[slab],
        out_specs=[slab, slab,
                   pl.BlockSpec((1, 1, 1), lambda b: (b, 0, 0)), slab],
        out_shape=[
            jax.ShapeDtypeStruct((B, V, F), jnp.float32),
            jax.ShapeDtypeStruct((B, V, F), jnp.int32),
            jax.ShapeDtypeStruct((B, 1, 1), jnp.float32),
            jax.ShapeDtypeStruct((B, V, F), jnp.float32),
        ],
        compiler_params=pltpu.CompilerParams(
            dimension_semantics=("parallel",)),
    )(stack_exp)
    return prob, cond, ent.reshape(B), sm


# speedup vs baseline: 2.0073x; 2.0073x over previous
import functools

import jax
import jax.numpy as jnp
from jax.experimental import pallas as pl
from jax.experimental.pallas import tpu as pltpu

_CH = 64


def _adj_body(K, x_ref, prob_ref, cond_ref, ent_ref, sm_ref):
    V, F = x_ref.shape[1], x_ref.shape[2]
    NCH = V // _CH

    row_iota = jax.lax.broadcasted_iota(jnp.int32, (_CH, F), 0)

    def pass_a(i, carry):
        acc = carry[0]
        ms = list(carry[1:])
        v = x_ref[0, pl.ds(i * _CH, _CH), :]
        acc = acc + v
        cur = v
        for j in range(K):
            hi = jnp.maximum(ms[j], cur)
            cur = jnp.minimum(ms[j], cur)
            ms[j] = hi
        return (acc, *ms)

    init_a = (jnp.zeros((_CH, F), jnp.float32),) + tuple(
        jnp.full((_CH, F), -jnp.inf, jnp.float32) for _ in range(K))
    res_a = jax.lax.fori_loop(0, NCH, pass_a, init_a)
    s = jnp.sum(res_a[0], axis=0, keepdims=True)
    cand_x = jnp.concatenate(res_a[1:], axis=0)
    sm_cand = jnp.clip(cand_x / (s + 1e-20), 0.001, 1.0 - 0.001)

    t = jnp.full((1, F), 2.0, jnp.float32)
    n = jnp.zeros((1, F), jnp.int32)
    c = jnp.zeros((1, F), jnp.int32)
    for _ in range(K):
        m = jnp.max(jnp.where(sm_cand < t, sm_cand, -1.0), axis=0,
                    keepdims=True)
        n_new = jnp.sum((sm_cand >= m).astype(jnp.int32), axis=0,
                        keepdims=True)
        upd = n < K
        c = jnp.where(upd, n, c)
        t = jnp.where(upd, m, t)
        n = jnp.where(upd, n_new, n)
    e = K - c

    def pass_b(i, carry):
        ent_acc = carry[0]
        js = list(carry[1:])
        xv = x_ref[0, pl.ds(i * _CH, _CH), :]
        smv = jnp.clip(xv / (s + 1e-20), 0.001, 1.0 - 0.001)
        sm_ref[0, pl.ds(i * _CH, _CH), :] = smv
        ent_acc = ent_acc - smv * jnp.log(smv)
        cur = jnp.where(smv == t, row_iota + i * _CH, V)
        for j in range(K):
            lo = jnp.minimum(js[j], cur)
            cur = jnp.maximum(js[j], cur)
            js[j] = lo
        return (ent_acc, *js)

    init_b = (jnp.zeros((_CH, F), jnp.float32),) + tuple(
        jnp.full((_CH, F), V, jnp.int32) for _ in range(K))
    res_b = jax.lax.fori_loop(0, NCH, pass_b, init_b)
    ent_ref[...] = (jnp.sum(res_b[0]) / F).reshape(1, 1, 1)
    cand_i = jnp.concatenate(res_b[1:], axis=0)

    s_last = jnp.full((1, F), -1, jnp.int32)
    last = jnp.full((1, F), -1, jnp.int32)
    for i in range(K):
        cnd = jnp.min(jnp.where(cand_i > last, cand_i, V), axis=0,
                      keepdims=True)
        s_last = jnp.where(i < e, cnd, s_last)
        last = cnd

    thr = 1.0 / (V * K)

    def pass_f(i, carry):
        sl = pl.ds(i * _CH, _CH)
        smv = sm_ref[0, sl, :]
        rmax = s_last - i * _CH
        mask = (smv > t) | ((smv == t) & (row_iota <= rmax))
        cond = (mask & (smv > thr)).astype(jnp.int32)
        cond_ref[0, sl, :] = cond
        prob_ref[0, sl, :] = jnp.where(cond == 1, jnp.log(smv), 0.0)
        return carry

    jax.lax.fori_loop(0, NCH, pass_f, 0)


def kernel(stack_exp):
    B, V, F = stack_exp.shape
    K = 8
    slab = pl.BlockSpec((1, V, F), lambda b: (b, 0, 0))
    prob, cond, ent, sm = pl.pallas_call(
        functools.partial(_adj_body, K),
        grid=(B,),
        in_specs=[slab],
        out_specs=[slab, slab,
                   pl.BlockSpec((1, 1, 1), lambda b: (b, 0, 0)), slab],
        out_shape=[
            jax.ShapeDtypeStruct((B, V, F), jnp.float32),
            jax.ShapeDtypeStruct((B, V, F), jnp.int32),
            jax.ShapeDtypeStruct((B, 1, 1), jnp.float32),
            jax.ShapeDtypeStruct((B, V, F), jnp.float32),
        ],
        compiler_params=pltpu.CompilerParams(
            dimension_semantics=("parallel",)),
    )(stack_exp)
    return prob, cond, ent.reshape(B), sm


# speedup vs baseline: 2.5699x; 1.2803x over previous
import functools

import jax
import jax.numpy as jnp
from jax.experimental import pallas as pl
from jax.experimental.pallas import tpu as pltpu

_CH = 32
_UNROLL = 4


def _adj_body(K, x_ref, prob_ref, cond_ref, ent_ref, sm_ref):
    V, F = x_ref.shape[1], x_ref.shape[2]
    NCH = V // (_CH * _UNROLL)

    row_iota = jax.lax.broadcasted_iota(jnp.int32, (_CH, F), 0)

    def pass_a(i, carry):
        acc = carry[0]
        ms = list(carry[1:])
        for u in range(_UNROLL):
            v = x_ref[0, pl.ds((i * _UNROLL + u) * _CH, _CH), :]
            acc = acc + v
            cur = v
            for j in range(K):
                hi = jnp.maximum(ms[j], cur)
                cur = jnp.minimum(ms[j], cur)
                ms[j] = hi
        return (acc, *ms)

    init_a = (jnp.zeros((_CH, F), jnp.float32),) + tuple(
        jnp.full((_CH, F), -jnp.inf, jnp.float32) for _ in range(K))
    res_a = jax.lax.fori_loop(0, NCH, pass_a, init_a)
    s = jnp.sum(res_a[0], axis=0, keepdims=True)
    rcp = 1.0 / (s + 1e-20)
    cand_x = jnp.concatenate(res_a[1:], axis=0)
    sm_cand = jnp.clip(cand_x * rcp, 0.001, 1.0 - 0.001)

    t = jnp.full((1, F), 2.0, jnp.float32)
    n = jnp.zeros((1, F), jnp.int32)
    c = jnp.zeros((1, F), jnp.int32)
    for _ in range(K):
        m = jnp.max(jnp.where(sm_cand < t, sm_cand, -1.0), axis=0,
                    keepdims=True)
        n_new = jnp.sum((sm_cand >= m).astype(jnp.int32), axis=0,
                        keepdims=True)
        upd = n < K
        c = jnp.where(upd, n, c)
        t = jnp.where(upd, m, t)
        n = jnp.where(upd, n_new, n)
    e = K - c

    def pass_b(i, carry):
        ent_acc = carry[0]
        js = list(carry[1:])
        for u in range(_UNROLL):
            base = (i * _UNROLL + u) * _CH
            xv = x_ref[0, pl.ds(base, _CH), :]
            smv = jnp.clip(xv * rcp, 0.001, 1.0 - 0.001)
            sm_ref[0, pl.ds(base, _CH), :] = smv
            ent_acc = ent_acc - smv * jnp.log(smv)
            cur = jnp.where(smv == t, row_iota + base, V)
            for j in range(K):
                lo = jnp.minimum(js[j], cur)
                cur = jnp.maximum(js[j], cur)
                js[j] = lo
        return (ent_acc, *js)

    init_b = (jnp.zeros((_CH, F), jnp.float32),) + tuple(
        jnp.full((_CH, F), V, jnp.int32) for _ in range(K))
    res_b = jax.lax.fori_loop(0, NCH, pass_b, init_b)
    ent_ref[...] = (jnp.sum(res_b[0]) / F).reshape(1, 1, 1)
    cand_i = jnp.concatenate(res_b[1:], axis=0)

    s_last = jnp.full((1, F), -1, jnp.int32)
    last = jnp.full((1, F), -1, jnp.int32)
    for i in range(K):
        cnd = jnp.min(jnp.where(cand_i > last, cand_i, V), axis=0,
                      keepdims=True)
        s_last = jnp.where(i < e, cnd, s_last)
        last = cnd

    thr = 1.0 / (V * K)

    def pass_f(i, carry):
        for u in range(_UNROLL):
            base = (i * _UNROLL + u) * _CH
            sl = pl.ds(base, _CH)
            smv = sm_ref[0, sl, :]
            rmax = s_last - base
            mask = (smv > t) | ((smv == t) & (row_iota <= rmax))
            cond = (mask & (smv > thr)).astype(jnp.int32)
            cond_ref[0, sl, :] = cond
            prob_ref[0, sl, :] = jnp.where(cond == 1, jnp.log(smv), 0.0)
        return carry

    jax.lax.fori_loop(0, NCH, pass_f, 0)


def kernel(stack_exp):
    B, V, F = stack_exp.shape
    K = 8
    slab = pl.BlockSpec((1, V, F), lambda b: (b, 0, 0))
    prob, cond, ent, sm = pl.pallas_call(
        functools.partial(_adj_body, K),
        grid=(B,),
        in_specs=[slab],
        out_specs=[slab, slab,
                   pl.BlockSpec((1, 1, 1), lambda b: (b, 0, 0)), slab],
        out_shape=[
            jax.ShapeDtypeStruct((B, V, F), jnp.float32),
            jax.ShapeDtypeStruct((B, V, F), jnp.int32),
            jax.ShapeDtypeStruct((B, 1, 1), jnp.float32),
            jax.ShapeDtypeStruct((B, V, F), jnp.float32),
        ],
        compiler_params=pltpu.CompilerParams(
            dimension_semantics=("parallel",)),
    )(stack_exp)
    return prob, cond, ent.reshape(B), sm


# speedup vs baseline: 2.5952x; 1.0098x over previous
import functools

import jax
import jax.numpy as jnp
from jax.experimental import pallas as pl
from jax.experimental.pallas import tpu as pltpu

_CH = 64
_UNROLL = 2


def _collapse(ms, desc):
    K = len(ms)
    sel_hi = jnp.maximum if desc else jnp.minimum
    while ms[0].shape[0] > 8:
        h = ms[0].shape[0] // 2
        a = [m[:h] for m in ms]
        b = [m[h:] for m in ms]
        part = [sel_hi(a[j], b[K - 1 - j]) for j in range(K)]
        for d in (4, 2, 1):
            nxt = list(part)
            for j in range(K):
                if j % (2 * d) < d:
                    hi = jnp.maximum(part[j], part[j + d])
                    lo = jnp.minimum(part[j], part[j + d])
                    nxt[j] = hi if desc else lo
                    nxt[j + d] = lo if desc else hi
            part = nxt
        ms = part
    return ms


def _adj_body(K, x_ref, prob_ref, cond_ref, ent_ref, sm_ref):
    V, F = x_ref.shape[1], x_ref.shape[2]
    NCH = V // (_CH * _UNROLL)

    row_iota = jax.lax.broadcasted_iota(jnp.int32, (_CH, F), 0)

    def pass_a(i, carry):
        acc = carry[0]
        ms = list(carry[1:])
        for u in range(_UNROLL):
            v = x_ref[0, pl.ds((i * _UNROLL + u) * _CH, _CH), :]
            acc = acc + v
            cur = v
            for j in range(K):
                hi = jnp.maximum(ms[j], cur)
                cur = jnp.minimum(ms[j], cur)
                ms[j] = hi
        return (acc, *ms)

    init_a = (jnp.zeros((_CH, F), jnp.float32),) + tuple(
        jnp.full((_CH, F), -jnp.inf, jnp.float32) for _ in range(K))
    res_a = jax.lax.fori_loop(0, NCH, pass_a, init_a)
    s = jnp.sum(res_a[0], axis=0, keepdims=True)
    rcp = 1.0 / (s + 1e-20)
    cand_x = jnp.concatenate(_collapse(list(res_a[1:]), True), axis=0)
    sm_cand = jnp.clip(cand_x * rcp, 0.001, 1.0 - 0.001)

    t = jnp.full((1, F), 2.0, jnp.float32)
    n = jnp.zeros((1, F), jnp.int32)
    c = jnp.zeros((1, F), jnp.int32)
    for _ in range(K):
        m = jnp.max(jnp.where(sm_cand < t, sm_cand, -1.0), axis=0,
                    keepdims=True)
        n_new = jnp.sum((sm_cand >= m).astype(jnp.int32), axis=0,
                        keepdims=True)
        upd = n < K
        c = jnp.where(upd, n, c)
        t = jnp.where(upd, m, t)
        n = jnp.where(upd, n_new, n)
    e = K - c

    def pass_b(i, carry):
        ent_acc = carry[0]
        js = list(carry[1:])
        for u in range(_UNROLL):
            base = (i * _UNROLL + u) * _CH
            xv = x_ref[0, pl.ds(base, _CH), :]
            smv = jnp.clip(xv * rcp, 0.001, 1.0 - 0.001)
            sm_ref[0, pl.ds(base, _CH), :] = smv
            ent_acc = ent_acc - smv * jnp.log(smv)
            cur = jnp.where(smv == t, row_iota + base, V)
            for j in range(K):
                lo = jnp.minimum(js[j], cur)
                cur = jnp.maximum(js[j], cur)
                js[j] = lo
        return (ent_acc, *js)

    init_b = (jnp.zeros((_CH, F), jnp.float32),) + tuple(
        jnp.full((_CH, F), V, jnp.int32) for _ in range(K))
    res_b = jax.lax.fori_loop(0, NCH, pass_b, init_b)
    ent_ref[...] = (jnp.sum(res_b[0]) / F).reshape(1, 1, 1)
    cand_i = jnp.concatenate(_collapse(list(res_b[1:]), False), axis=0)

    s_last = jnp.full((1, F), -1, jnp.int32)
    last = jnp.full((1, F), -1, jnp.int32)
    for i in range(K):
        cnd = jnp.min(jnp.where(cand_i > last, cand_i, V), axis=0,
                      keepdims=True)
        s_last = jnp.where(i < e, cnd, s_last)
        last = cnd

    thr = 1.0 / (V * K)

    def pass_f(i, carry):
        for u in range(_UNROLL):
            base = (i * _UNROLL + u) * _CH
            sl = pl.ds(base, _CH)
            smv = sm_ref[0, sl, :]
            rmax = s_last - base
            mask = (smv > t) | ((smv == t) & (row_iota <= rmax))
            cond = (mask & (smv > thr)).astype(jnp.int32)
            cond_ref[0, sl, :] = cond
            prob_ref[0, sl, :] = jnp.where(cond == 1, jnp.log(smv), 0.0)
        return carry

    jax.lax.fori_loop(0, NCH, pass_f, 0)


def kernel(stack_exp):
    B, V, F = stack_exp.shape
    K = 8
    slab = pl.BlockSpec((1, V, F), lambda b: (b, 0, 0))
    prob, cond, ent, sm = pl.pallas_call(
        functools.partial(_adj_body, K),
        grid=(B,),
        in_specs=[slab],
        out_specs=[slab, slab,
                   pl.BlockSpec((1, 1, 1), lambda b: (b, 0, 0)), slab],
        out_shape=[
            jax.ShapeDtypeStruct((B, V, F), jnp.float32),
            jax.ShapeDtypeStruct((B, V, F), jnp.int32),
            jax.ShapeDtypeStruct((B, 1, 1), jnp.float32),
            jax.ShapeDtypeStruct((B, V, F), jnp.float32),
        ],
        compiler_params=pltpu.CompilerParams(
            dimension_semantics=("parallel",)),
    )(stack_exp)
    return prob, cond, ent.reshape(B), sm
